# baseline (speedup 1.0000x reference)
import functools

import jax
import jax.numpy as jnp
from jax import lax
from jax.experimental import pallas as pl
from jax.experimental.pallas import tpu as pltpu
from jax.experimental.pallas import tpu_sc as plsc

N = 10000
E = 320000
IN_D = 128
HID = 64
OUT_D = 2
K0 = 80
K1 = 16
NC = 2
NS = 16
NW = NC * NS
EPW = E // NW
CH = 200
NCHUNK = EPW // CH
RPT = 624
RTAIL = N - NS * RPT
R = 1000
G = N // R


def _sc_scatter(K):
    mesh = plsc.VectorSubcoreMesh(core_axis_name="c", subcore_axis_name="s")

    @functools.partial(
        pl.kernel,
        out_type=jax.ShapeDtypeStruct((NC, N, K), jnp.float32),
        mesh=mesh,
        scratch_types=[
            pltpu.VMEM((NCHUNK, CH), jnp.int32),
            pltpu.VMEM((NCHUNK, CH), jnp.int32),
            pltpu.VMEM((2, CH, K), jnp.float32),
            pltpu.VMEM_SHARED((N, K), jnp.float32),
            [pltpu.SemaphoreType.DMA] * 2,
            [pltpu.SemaphoreType.DMA] * 2,
        ],
        compiler_params=pltpu.CompilerParams(use_tc_tiling_on_sc=False),
    )
    def k(p_hbm, src_hbm, dst_hbm, zero_hbm, out_hbm, srcs, dsts, rows,
          agg_sh, sem_g, sem_s):
        c = lax.axis_index("c")
        s = lax.axis_index("s")
        wid = c * NS + s
        row0 = s * RPT
        pltpu.sync_copy(zero_hbm.at[pl.ds(row0, RPT)],
                        agg_sh.at[pl.ds(row0, RPT)])
        @pl.when(s == 0)
        def _():
            pltpu.sync_copy(zero_hbm.at[pl.ds(NS * RPT, RTAIL)],
                            agg_sh.at[pl.ds(NS * RPT, RTAIL)])
        pltpu.sync_copy(src_hbm.at[wid], srcs)
        pltpu.sync_copy(dst_hbm.at[wid], dsts)
        plsc.subcore_barrier()

        def issue_g(i, b):
            pltpu.async_copy(p_hbm.at[srcs.at[i]], rows.at[b], sem_g[b])

        def wait_g(b):
            pltpu.make_async_copy(p_hbm.at[srcs.at[0]], rows.at[b],
                                  sem_g[b]).wait()

        def issue_s(i, b):
            pltpu.async_copy(rows.at[b], agg_sh.at[dsts.at[i]], sem_s[b],
                             add=True)

        def wait_s(b):
            pltpu.make_async_copy(rows.at[b], agg_sh.at[dsts.at[0]],
                                  sem_s[b]).wait()

        issue_g(0, 0)
        issue_g(1, 1)
        wait_g(0)
        issue_s(0, 0)

        def body(j, carry):
            for t in (0, 1):
                i = 2 * j + 1 + t
                b = 1 - t
                wait_g(b)
                issue_s(i, b)
                wait_s(1 - b)
                issue_g(i + 1, 1 - b)
            return carry

        lax.fori_loop(0, (NCHUNK - 2) // 2, body, 0)
        wait_g(1)
        issue_s(NCHUNK - 1, 1)
        wait_s(0)
        wait_s(1)
        plsc.subcore_barrier()
        pltpu.sync_copy(agg_sh.at[pl.ds(row0, RPT)],
                        out_hbm.at[c].at[pl.ds(row0, RPT)])
        @pl.when(s == 0)
        def _():
            pltpu.sync_copy(agg_sh.at[pl.ds(NS * RPT, RTAIL)],
                            out_hbm.at[c].at[pl.ds(NS * RPT, RTAIL)])

    return k


def _tc1(x, wd, wp, bd, bp):
    def body(x_ref, wd_ref, wp_ref, bd_ref, bp_ref, d_ref, p_ref):
        xb = x_ref[...]
        d_ref[...] = (jnp.dot(xb, wd_ref[...],
                              preferred_element_type=jnp.float32)
                      + bd_ref[...])
        p_ref[...] = (jnp.dot(xb, wp_ref[...],
                              preferred_element_type=jnp.float32)
                      + bp_ref[...])

    return pl.pallas_call(
        body,
        grid=(G,),
        in_specs=[
            pl.BlockSpec((R, IN_D), lambda i: (i, 0)),
            pl.BlockSpec((IN_D, HID), lambda i: (0, 0)),
            pl.BlockSpec((IN_D, K0), lambda i: (0, 0)),
            pl.BlockSpec((1, HID), lambda i: (0, 0)),
            pl.BlockSpec((1, K0), lambda i: (0, 0)),
        ],
        out_specs=[
            pl.BlockSpec((R, HID), lambda i: (i, 0)),
            pl.BlockSpec((R, K0), lambda i: (i, 0)),
        ],
        out_shape=[
            jax.ShapeDtypeStruct((N, HID), jnp.float32),
            jax.ShapeDtypeStruct((N, K0), jnp.float32),
        ],
    )(x, wd, wp, bd, bp)


def _tc2(d0, agg0p, w2l, b2, w2rp):
    def body(d_ref, a_ref, wl_ref, b2_ref, wr_ref, d2_ref, q_ref, rc_ref):
        t = a_ref[...]
        a = t[0] + t[1]
        cnt = a[:, HID:HID + 1]
        rc = 1.0 / jnp.maximum(cnt, 1.0)
        h = jnp.maximum(d_ref[...] + a[:, :HID] * rc, 0.0)
        d2_ref[...] = (jnp.dot(h, wl_ref[...],
                               preferred_element_type=jnp.float32)
                       + b2_ref[...])
        q_ref[...] = jnp.dot(h, wr_ref[...],
                             preferred_element_type=jnp.float32)
        rc_ref[...] = jnp.broadcast_to(rc, (R, K1))

    return pl.pallas_call(
        body,
        grid=(G,),
        in_specs=[
            pl.BlockSpec((R, HID), lambda i: (i, 0)),
            pl.BlockSpec((NC, R, K0), lambda i: (0, i, 0)),
            pl.BlockSpec((HID, OUT_D), lambda i: (0, 0)),
            pl.BlockSpec((1, OUT_D), lambda i: (0, 0)),
            pl.BlockSpec((HID, K1), lambda i: (0, 0)),
        ],
        out_specs=[
            pl.BlockSpec((R, OUT_D), lambda i: (i, 0)),
            pl.BlockSpec((R, K1), lambda i: (i, 0)),
            pl.BlockSpec((R, K1), lambda i: (i, 0)),
        ],
        out_shape=[
            jax.ShapeDtypeStruct((N, OUT_D), jnp.float32),
            jax.ShapeDtypeStruct((N, K1), jnp.float32),
            jax.ShapeDtypeStruct((N, K1), jnp.float32),
        ],
    )(d0, agg0p, w2l, b2, w2rp)


def _tc3(d2, agg1p, rc16):
    def body(d2_ref, a_ref, rc_ref, o_ref):
        t = a_ref[...]
        a = t[0] + t[1]
        o_ref[...] = d2_ref[...] + a[:, :OUT_D] * rc_ref[...][:, :OUT_D]

    return pl.pallas_call(
        body,
        grid=(G,),
        in_specs=[
            pl.BlockSpec((R, OUT_D), lambda i: (i, 0)),
            pl.BlockSpec((NC, R, K1), lambda i: (0, i, 0)),
            pl.BlockSpec((R, K1), lambda i: (i, 0)),
        ],
        out_specs=pl.BlockSpec((R, OUT_D), lambda i: (i, 0)),
        out_shape=jax.ShapeDtypeStruct((N, OUT_D), jnp.float32),
    )(d2, agg1p, rc16)


def kernel(x, edge_index, W1, b1, W2, b2):
    src = edge_index[0]
    dst = edge_index[1]
    wd = W1[:, :IN_D].T
    w1r = W1[:, IN_D:].T
    wp = jnp.concatenate(
        [w1r, jnp.zeros((IN_D, K0 - HID), jnp.float32)], axis=1)
    bd = b1.reshape(1, HID)
    bp = jnp.zeros((1, K0), jnp.float32).at[0, HID].set(1.0)
    w2l = W2[:, :HID].T
    w2r = W2[:, HID:].T
    w2rp = jnp.concatenate(
        [w2r, jnp.zeros((HID, K1 - OUT_D), jnp.float32)], axis=1)
    b2r = b2.reshape(1, OUT_D)
    zeros0 = jnp.zeros((N, K0), jnp.float32)
    zeros1 = jnp.zeros((N, K1), jnp.float32)

    src3 = src.reshape(NW, NCHUNK, CH)
    dst3 = dst.reshape(NW, NCHUNK, CH)

    d0, p0p = _tc1(x, wd, wp, bd, bp)
    agg0p = _sc_scatter(K0)(p0p, src3, dst3, zeros0)
    d2, q1p, rc16 = _tc2(d0, agg0p, w2l, b2r, w2rp)
    agg1p = _sc_scatter(K1)(q1p, src3, dst3, zeros1)
    return _tc3(d2, agg1p, rc16)


# speedup vs baseline: 1.1214x; 1.1214x over previous
import functools

import jax
import jax.numpy as jnp
from jax import lax
from jax.experimental import pallas as pl
from jax.experimental.pallas import tpu as pltpu
from jax.experimental.pallas import tpu_sc as plsc

N = 10000
E = 320000
IN_D = 128
HID = 64
OUT_D = 2
K0 = 80
K1 = 16
NC = 2
NS = 16
NW = NC * NS
EPW = E // NW
CH = 200
NCHUNK = EPW // CH
RPT = 624
RTAIL = N - NS * RPT
R = 1000
G = N // R


def _sc_scatter(K, stage_p=False):
    mesh = plsc.VectorSubcoreMesh(core_axis_name="c", subcore_axis_name="s")

    scratch = [
        pltpu.VMEM((NCHUNK, CH), jnp.int32),
        pltpu.VMEM((NCHUNK, CH), jnp.int32),
        pltpu.VMEM((2, CH, K), jnp.float32),
        pltpu.VMEM_SHARED((N, K), jnp.float32),
        [pltpu.SemaphoreType.DMA] * 2,
        [pltpu.SemaphoreType.DMA] * 2,
    ]
    if stage_p:
        scratch.append(pltpu.VMEM_SHARED((N, K), jnp.float32))

    @functools.partial(
        pl.kernel,
        out_type=jax.ShapeDtypeStruct((NC, N, K), jnp.float32),
        mesh=mesh,
        scratch_types=scratch,
        compiler_params=pltpu.CompilerParams(use_tc_tiling_on_sc=False),
    )
    def k(p_hbm, src_hbm, dst_hbm, zero_hbm, out_hbm, srcs, dsts, rows,
          agg_sh, sem_g, sem_s, *maybe_psh):
        c = lax.axis_index("c")
        s = lax.axis_index("s")
        wid = c * NS + s
        row0 = s * RPT
        pltpu.sync_copy(zero_hbm.at[pl.ds(row0, RPT)],
                        agg_sh.at[pl.ds(row0, RPT)])
        @pl.when(s == 0)
        def _():
            pltpu.sync_copy(zero_hbm.at[pl.ds(NS * RPT, RTAIL)],
                            agg_sh.at[pl.ds(NS * RPT, RTAIL)])
        if stage_p:
            p_src = maybe_psh[0]
            pltpu.sync_copy(p_hbm.at[pl.ds(row0, RPT)],
                            p_src.at[pl.ds(row0, RPT)])
            @pl.when(s == 0)
            def _():
                pltpu.sync_copy(p_hbm.at[pl.ds(NS * RPT, RTAIL)],
                                p_src.at[pl.ds(NS * RPT, RTAIL)])
        else:
            p_src = p_hbm
        pltpu.sync_copy(src_hbm.at[wid], srcs)
        pltpu.sync_copy(dst_hbm.at[wid], dsts)
        plsc.subcore_barrier()

        def issue_g(i, b):
            pltpu.async_copy(p_src.at[srcs.at[i]], rows.at[b], sem_g[b])

        def wait_g(b):
            pltpu.make_async_copy(p_src.at[srcs.at[0]], rows.at[b],
                                  sem_g[b]).wait()

        def issue_s(i, b):
            pltpu.async_copy(rows.at[b], agg_sh.at[dsts.at[i]], sem_s[b],
                             add=True)

        def wait_s(b):
            pltpu.make_async_copy(rows.at[b], agg_sh.at[dsts.at[0]],
                                  sem_s[b]).wait()

        issue_g(0, 0)
        issue_g(1, 1)
        wait_g(0)
        issue_s(0, 0)

        def body(j, carry):
            for t in (0, 1):
                i = 2 * j + 1 + t
                b = 1 - t
                wait_g(b)
                issue_s(i, b)
                wait_s(1 - b)
                issue_g(i + 1, 1 - b)
            return carry

        lax.fori_loop(0, (NCHUNK - 2) // 2, body, 0)
        wait_g(1)
        issue_s(NCHUNK - 1, 1)
        wait_s(0)
        wait_s(1)
        plsc.subcore_barrier()
        pltpu.sync_copy(agg_sh.at[pl.ds(row0, RPT)],
                        out_hbm.at[c].at[pl.ds(row0, RPT)])
        @pl.when(s == 0)
        def _():
            pltpu.sync_copy(agg_sh.at[pl.ds(NS * RPT, RTAIL)],
                            out_hbm.at[c].at[pl.ds(NS * RPT, RTAIL)])

    return k


def _tc1(x, wd, wp, bd, bp):
    def body(x_ref, wd_ref, wp_ref, bd_ref, bp_ref, d_ref, p_ref):
        xb = x_ref[...]
        d_ref[...] = (jnp.dot(xb, wd_ref[...],
                              preferred_element_type=jnp.float32)
                      + bd_ref[...])
        p_ref[...] = (jnp.dot(xb, wp_ref[...],
                              preferred_element_type=jnp.float32)
                      + bp_ref[...])

    return pl.pallas_call(
        body,
        grid=(G,),
        in_specs=[
            pl.BlockSpec((R, IN_D), lambda i: (i, 0)),
            pl.BlockSpec((IN_D, HID), lambda i: (0, 0)),
            pl.BlockSpec((IN_D, K0), lambda i: (0, 0)),
            pl.BlockSpec((1, HID), lambda i: (0, 0)),
            pl.BlockSpec((1, K0), lambda i: (0, 0)),
        ],
        out_specs=[
            pl.BlockSpec((R, HID), lambda i: (i, 0)),
            pl.BlockSpec((R, K0), lambda i: (i, 0)),
        ],
        out_shape=[
            jax.ShapeDtypeStruct((N, HID), jnp.float32),
            jax.ShapeDtypeStruct((N, K0), jnp.float32),
        ],
    )(x, wd, wp, bd, bp)


def _tc2(d0, agg0p, w2l, b2, w2rp):
    def body(d_ref, a_ref, wl_ref, b2_ref, wr_ref, d2_ref, q_ref, rc_ref):
        t = a_ref[...]
        a = t[0] + t[1]
        cnt = a[:, HID:HID + 1]
        rc = 1.0 / jnp.maximum(cnt, 1.0)
        h = jnp.maximum(d_ref[...] + a[:, :HID] * rc, 0.0)
        d2_ref[...] = (jnp.dot(h, wl_ref[...],
                               preferred_element_type=jnp.float32)
                       + b2_ref[...])
        q_ref[...] = jnp.dot(h, wr_ref[...],
                             preferred_element_type=jnp.float32)
        rc_ref[...] = jnp.broadcast_to(rc, (R, K1))

    return pl.pallas_call(
        body,
        grid=(G,),
        in_specs=[
            pl.BlockSpec((R, HID), lambda i: (i, 0)),
            pl.BlockSpec((NC, R, K0), lambda i: (0, i, 0)),
            pl.BlockSpec((HID, OUT_D), lambda i: (0, 0)),
            pl.BlockSpec((1, OUT_D), lambda i: (0, 0)),
            pl.BlockSpec((HID, K1), lambda i: (0, 0)),
        ],
        out_specs=[
            pl.BlockSpec((R, OUT_D), lambda i: (i, 0)),
            pl.BlockSpec((R, K1), lambda i: (i, 0)),
            pl.BlockSpec((R, K1), lambda i: (i, 0)),
        ],
        out_shape=[
            jax.ShapeDtypeStruct((N, OUT_D), jnp.float32),
            jax.ShapeDtypeStruct((N, K1), jnp.float32),
            jax.ShapeDtypeStruct((N, K1), jnp.float32),
        ],
    )(d0, agg0p, w2l, b2, w2rp)


def _tc3(d2, agg1p, rc16):
    def body(d2_ref, a_ref, rc_ref, o_ref):
        t = a_ref[...]
        a = t[0] + t[1]
        o_ref[...] = d2_ref[...] + a[:, :OUT_D] * rc_ref[...][:, :OUT_D]

    return pl.pallas_call(
        body,
        grid=(G,),
        in_specs=[
            pl.BlockSpec((R, OUT_D), lambda i: (i, 0)),
            pl.BlockSpec((NC, R, K1), lambda i: (0, i, 0)),
            pl.BlockSpec((R, K1), lambda i: (i, 0)),
        ],
        out_specs=pl.BlockSpec((R, OUT_D), lambda i: (i, 0)),
        out_shape=jax.ShapeDtypeStruct((N, OUT_D), jnp.float32),
    )(d2, agg1p, rc16)


def kernel(x, edge_index, W1, b1, W2, b2):
    src = edge_index[0]
    dst = edge_index[1]
    wd = W1[:, :IN_D].T
    w1r = W1[:, IN_D:].T
    wp = jnp.concatenate(
        [w1r, jnp.zeros((IN_D, K0 - HID), jnp.float32)], axis=1)
    bd = b1.reshape(1, HID)
    bp = jnp.zeros((1, K0), jnp.float32).at[0, HID].set(1.0)
    w2l = W2[:, :HID].T
    w2r = W2[:, HID:].T
    w2rp = jnp.concatenate(
        [w2r, jnp.zeros((HID, K1 - OUT_D), jnp.float32)], axis=1)
    b2r = b2.reshape(1, OUT_D)
    zeros0 = jnp.zeros((N, K0), jnp.float32)
    zeros1 = jnp.zeros((N, K1), jnp.float32)

    src3 = src.reshape(NW, NCHUNK, CH)
    dst3 = dst.reshape(NW, NCHUNK, CH)

    d0, p0p = _tc1(x, wd, wp, bd, bp)
    agg0p = _sc_scatter(K0)(p0p, src3, dst3, zeros0)
    d2, q1p, rc16 = _tc2(d0, agg0p, w2l, b2r, w2rp)
    agg1p = _sc_scatter(K1, stage_p=True)(q1p, src3, dst3, zeros1)
    return _tc3(d2, agg1p, rc16)


# speedup vs baseline: 1.1280x; 1.0059x over previous
import functools

import jax
import jax.numpy as jnp
from jax import lax
from jax.experimental import pallas as pl
from jax.experimental.pallas import tpu as pltpu
from jax.experimental.pallas import tpu_sc as plsc

N = 10000
E = 320000
IN_D = 128
HID = 64
OUT_D = 2
K0 = 72
K1 = 16
NC = 2
NS = 16
NW = NC * NS
EPW = E // NW
CH = 200
NCHUNK = EPW // CH
RPT = 624
RTAIL = N - NS * RPT
R = 1000
G = N // R


def _sc_scatter(K, stage_p=False):
    mesh = plsc.VectorSubcoreMesh(core_axis_name="c", subcore_axis_name="s")

    scratch = [
        pltpu.VMEM((NCHUNK, CH), jnp.int32),
        pltpu.VMEM((NCHUNK, CH), jnp.int32),
        pltpu.VMEM((2, CH, K), jnp.float32),
        pltpu.VMEM_SHARED((N, K), jnp.float32),
        [pltpu.SemaphoreType.DMA] * 2,
        [pltpu.SemaphoreType.DMA] * 2,
    ]
    if stage_p:
        scratch.append(pltpu.VMEM_SHARED((N, K), jnp.float32))

    @functools.partial(
        pl.kernel,
        out_type=jax.ShapeDtypeStruct((NC, N, K), jnp.float32),
        mesh=mesh,
        scratch_types=scratch,
        compiler_params=pltpu.CompilerParams(use_tc_tiling_on_sc=False),
    )
    def k(p_hbm, src_hbm, dst_hbm, zero_hbm, out_hbm, srcs, dsts, rows,
          agg_sh, sem_g, sem_s, *maybe_psh):
        c = lax.axis_index("c")
        s = lax.axis_index("s")
        wid = c * NS + s
        row0 = s * RPT
        pltpu.sync_copy(zero_hbm.at[pl.ds(row0, RPT)],
                        agg_sh.at[pl.ds(row0, RPT)])
        @pl.when(s == 0)
        def _():
            pltpu.sync_copy(zero_hbm.at[pl.ds(NS * RPT, RTAIL)],
                            agg_sh.at[pl.ds(NS * RPT, RTAIL)])
        if stage_p:
            p_src = maybe_psh[0]
            pltpu.sync_copy(p_hbm.at[pl.ds(row0, RPT)],
                            p_src.at[pl.ds(row0, RPT)])
            @pl.when(s == 0)
            def _():
                pltpu.sync_copy(p_hbm.at[pl.ds(NS * RPT, RTAIL)],
                                p_src.at[pl.ds(NS * RPT, RTAIL)])
        else:
            p_src = p_hbm
        pltpu.sync_copy(src_hbm.at[wid], srcs)
        pltpu.sync_copy(dst_hbm.at[wid], dsts)
        plsc.subcore_barrier()

        def issue_g(i, b):
            pltpu.async_copy(p_src.at[srcs.at[i]], rows.at[b], sem_g[b])

        def wait_g(b):
            pltpu.make_async_copy(p_src.at[srcs.at[0]], rows.at[b],
                                  sem_g[b]).wait()

        def issue_s(i, b):
            pltpu.async_copy(rows.at[b], agg_sh.at[dsts.at[i]], sem_s[b],
                             add=True)

        def wait_s(b):
            pltpu.make_async_copy(rows.at[b], agg_sh.at[dsts.at[0]],
                                  sem_s[b]).wait()

        issue_g(0, 0)
        issue_g(1, 1)
        wait_g(0)
        issue_s(0, 0)

        def body(j, carry):
            for t in (0, 1):
                i = 2 * j + 1 + t
                b = 1 - t
                wait_g(b)
                issue_s(i, b)
                wait_s(1 - b)
                issue_g(i + 1, 1 - b)
            return carry

        lax.fori_loop(0, (NCHUNK - 2) // 2, body, 0)
        wait_g(1)
        issue_s(NCHUNK - 1, 1)
        wait_s(0)
        wait_s(1)
        plsc.subcore_barrier()
        pltpu.sync_copy(agg_sh.at[pl.ds(row0, RPT)],
                        out_hbm.at[c].at[pl.ds(row0, RPT)])
        @pl.when(s == 0)
        def _():
            pltpu.sync_copy(agg_sh.at[pl.ds(NS * RPT, RTAIL)],
                            out_hbm.at[c].at[pl.ds(NS * RPT, RTAIL)])

    return k


def _tc1(x, wd, wp, bd, bp):
    def body(x_ref, wd_ref, wp_ref, bd_ref, bp_ref, d_ref, p_ref):
        xb = x_ref[...]
        d_ref[...] = (jnp.dot(xb, wd_ref[...],
                              preferred_element_type=jnp.float32)
                      + bd_ref[...])
        p_ref[...] = (jnp.dot(xb, wp_ref[...],
                              preferred_element_type=jnp.float32)
                      + bp_ref[...])

    return pl.pallas_call(
        body,
        grid=(G,),
        in_specs=[
            pl.BlockSpec((R, IN_D), lambda i: (i, 0)),
            pl.BlockSpec((IN_D, HID), lambda i: (0, 0)),
            pl.BlockSpec((IN_D, K0), lambda i: (0, 0)),
            pl.BlockSpec((1, HID), lambda i: (0, 0)),
            pl.BlockSpec((1, K0), lambda i: (0, 0)),
        ],
        out_specs=[
            pl.BlockSpec((R, HID), lambda i: (i, 0)),
            pl.BlockSpec((R, K0), lambda i: (i, 0)),
        ],
        out_shape=[
            jax.ShapeDtypeStruct((N, HID), jnp.float32),
            jax.ShapeDtypeStruct((N, K0), jnp.float32),
        ],
    )(x, wd, wp, bd, bp)


def _tc2(d0, agg0p, w2l, b2, w2rp):
    def body(d_ref, a_ref, wl_ref, b2_ref, wr_ref, d2_ref, q_ref, rc_ref):
        t = a_ref[...]
        a = t[0] + t[1]
        cnt = a[:, HID:HID + 1]
        rc = 1.0 / jnp.maximum(cnt, 1.0)
        h = jnp.maximum(d_ref[...] + a[:, :HID] * rc, 0.0)
        d2_ref[...] = (jnp.dot(h, wl_ref[...],
                               preferred_element_type=jnp.float32)
                       + b2_ref[...])
        q_ref[...] = jnp.dot(h, wr_ref[...],
                             preferred_element_type=jnp.float32)
        rc_ref[...] = jnp.broadcast_to(rc, (R, K1))

    return pl.pallas_call(
        body,
        grid=(G,),
        in_specs=[
            pl.BlockSpec((R, HID), lambda i: (i, 0)),
            pl.BlockSpec((NC, R, K0), lambda i: (0, i, 0)),
            pl.BlockSpec((HID, OUT_D), lambda i: (0, 0)),
            pl.BlockSpec((1, OUT_D), lambda i: (0, 0)),
            pl.BlockSpec((HID, K1), lambda i: (0, 0)),
        ],
        out_specs=[
            pl.BlockSpec((R, OUT_D), lambda i: (i, 0)),
            pl.BlockSpec((R, K1), lambda i: (i, 0)),
            pl.BlockSpec((R, K1), lambda i: (i, 0)),
        ],
        out_shape=[
            jax.ShapeDtypeStruct((N, OUT_D), jnp.float32),
            jax.ShapeDtypeStruct((N, K1), jnp.float32),
            jax.ShapeDtypeStruct((N, K1), jnp.float32),
        ],
    )(d0, agg0p, w2l, b2, w2rp)


def _tc3(d2, agg1p, rc16):
    def body(d2_ref, a_ref, rc_ref, o_ref):
        t = a_ref[...]
        a = t[0] + t[1]
        o_ref[...] = d2_ref[...] + a[:, :OUT_D] * rc_ref[...][:, :OUT_D]

    return pl.pallas_call(
        body,
        grid=(G,),
        in_specs=[
            pl.BlockSpec((R, OUT_D), lambda i: (i, 0)),
            pl.BlockSpec((NC, R, K1), lambda i: (0, i, 0)),
            pl.BlockSpec((R, K1), lambda i: (i, 0)),
        ],
        out_specs=pl.BlockSpec((R, OUT_D), lambda i: (i, 0)),
        out_shape=jax.ShapeDtypeStruct((N, OUT_D), jnp.float32),
    )(d2, agg1p, rc16)


def kernel(x, edge_index, W1, b1, W2, b2):
    src = edge_index[0]
    dst = edge_index[1]
    wd = W1[:, :IN_D].T
    w1r = W1[:, IN_D:].T
    wp = jnp.concatenate(
        [w1r, jnp.zeros((IN_D, K0 - HID), jnp.float32)], axis=1)
    bd = b1.reshape(1, HID)
    bp = jnp.zeros((1, K0), jnp.float32).at[0, HID].set(1.0)
    w2l = W2[:, :HID].T
    w2r = W2[:, HID:].T
    w2rp = jnp.concatenate(
        [w2r, jnp.zeros((HID, K1 - OUT_D), jnp.float32)], axis=1)
    b2r = b2.reshape(1, OUT_D)
    zeros0 = jnp.zeros((N, K0), jnp.float32)
    zeros1 = jnp.zeros((N, K1), jnp.float32)

    src3 = src.reshape(NW, NCHUNK, CH)
    dst3 = dst.reshape(NW, NCHUNK, CH)

    d0, p0p = _tc1(x, wd, wp, bd, bp)
    agg0p = _sc_scatter(K0)(p0p, src3, dst3, zeros0)
    d2, q1p, rc16 = _tc2(d0, agg0p, w2l, b2r, w2rp)
    agg1p = _sc_scatter(K1, stage_p=True)(q1p, src3, dst3, zeros1)
    return _tc3(d2, agg1p, rc16)
